# baseline (speedup 1.0000x reference)
import functools

import jax
import jax.numpy as jnp
from jax import lax
from jax.experimental import pallas as pl
from jax.experimental.pallas import tpu as pltpu
from jax.experimental.pallas import tpu_sc as plsc

N = 10000
C = 128
K = 16
R = 4
KC = 32
NPAD = 10240
BQ = 512
NBLK = NPAD // BQ
BN = 1024

_SC_NC = 2
_SC_NS = 16
_SC_W = _SC_NC * _SC_NS
_QPW = NPAD // _SC_W
_BATCH = 8
_NBATCH = _QPW // _BATCH


_MP = 6
KCX = KC - 1


def _knn_body(q8_ref, posT_ref, sqc_ref, sqr_ref, out_ref, d_ref, v_ref, j_ref):
    i = pl.program_id(0)
    inf = jnp.float32(jnp.inf)
    big = jnp.int32(0x7FFFFFFF)
    d = sqc_ref[:, :] + sqr_ref[:, :] - 2.0 * jnp.dot(
        q8_ref[:, :], posT_ref[:, :], preferred_element_type=jnp.float32)
    jidx = lax.broadcasted_iota(jnp.int32, (BQ, NPAD), 1)
    qidx = i * BQ + lax.broadcasted_iota(jnp.int32, (BQ, NPAD), 0)
    d_ref[:, :] = jnp.where((jidx == qidx) | (jidx >= N), inf, d)

    lane = lax.broadcasted_iota(jnp.int32, (BQ, 128), 1)
    vals = [jnp.full((BQ, 128), inf, jnp.float32) for _ in range(_MP)]
    js = [jnp.zeros((BQ, 128), jnp.int32) for _ in range(_MP)]
    for t in range(NPAD // 128):
        v = d_ref[:, t * 128:(t + 1) * 128]
        j = t * 128 + lane
        for lvl in range(_MP):
            c = v < vals[lvl]
            nv = jnp.where(c, v, vals[lvl])
            pv = jnp.where(c, vals[lvl], v)
            nj = jnp.where(c, j, js[lvl])
            pj = jnp.where(c, js[lvl], j)
            vals[lvl], js[lvl], v, j = nv, nj, pv, pj
    v_ref[:, :] = jnp.concatenate(vals, axis=1)
    j_ref[:, :] = jnp.concatenate(js, axis=1)
    vlast = vals[_MP - 1]

    v32 = None
    for k in range(KCX):
        V = v_ref[:, :]
        Jv = j_ref[:, :]
        m = jnp.min(V, axis=1)
        am = jnp.min(jnp.where(V == m[:, None], Jv, big), axis=1)
        out_ref[:, k] = am
        v_ref[:, :] = jnp.where(Jv == am[:, None], inf, V)
        if k == KCX - 1:
            v32 = m

    anyfail = jnp.any(vlast <= v32[:, None])

    @pl.when(anyfail)
    def _fallback():
        for k in range(KCX):
            dcur = d_ref[:, :]
            m = jnp.min(dcur, axis=1)
            am = jnp.min(jnp.where(dcur == m[:, None], jidx, big), axis=1)
            out_ref[:, k] = am
            d_ref[:, :] = jnp.where(jidx == am[:, None], inf, dcur)


_knn_call = pl.pallas_call(
    _knn_body,
    grid=(NBLK,),
    in_specs=[
        pl.BlockSpec((BQ, 8), lambda i: (i, 0)),
        pl.BlockSpec((8, NPAD), lambda i: (0, 0)),
        pl.BlockSpec((BQ, 1), lambda i: (i, 0)),
        pl.BlockSpec((1, NPAD), lambda i: (0, 0)),
    ],
    out_specs=pl.BlockSpec((BQ, KC), lambda i: (i, 0)),
    out_shape=jax.ShapeDtypeStruct((NPAD, KC), jnp.int32),
    scratch_shapes=[
        pltpu.VMEM((BQ, NPAD), jnp.float32),
        pltpu.VMEM((BQ, _MP * 128), jnp.float32),
        pltpu.VMEM((BQ, _MP * 128), jnp.int32),
    ],
)


_B2 = 4
_NB2 = _QPW // _B2


@functools.lru_cache(maxsize=None)
def _segmax(D, two):
    mesh = plsc.VectorSubcoreMesh(core_axis_name="c", subcore_axis_name="s")
    n_out = 2 if two else 1
    out_type = [jax.ShapeDtypeStruct((NPAD, D), jnp.float32)] * n_out

    @functools.partial(
        pl.kernel,
        out_type=out_type if two else out_type[0],
        mesh=mesh,
        scratch_types=[
            pltpu.VMEM((_QPW * K,), jnp.int32),
            pltpu.VMEM((_B2 * K, D), jnp.float32),
            pltpu.VMEM((_B2 * K, D), jnp.float32),
            pltpu.VMEM((_B2, D), jnp.float32),
            pltpu.SemaphoreType.DMA,
            pltpu.SemaphoreType.DMA,
        ],
    )
    def seg(*refs):
        if two:
            t0, i0, t1, i1, o0, o1, idxw, r0, r1, outb, s0, s1 = refs
            work = ((t0, i0, o0), (t1, i1, o1))
        else:
            t0, i0, o0, idxw, r0, r1, outb, s0, s1 = refs
            work = ((t0, i0, o0),)
        rows = (r0, r1)
        sems = (s0, s1)
        wid = lax.axis_index("s") * _SC_NC + lax.axis_index("c")

        for table, idx, out in work:
            pltpu.sync_copy(idx.at[pl.ds(wid * _QPW * K, _QPW * K)], idxw)

            def start(b, u):
                pltpu.async_copy(
                    table.at[idxw.at[pl.ds(b * (_B2 * K), _B2 * K)]],
                    rows[u], sems[u])

            def do_batch(b, u, issue_nb):
                pltpu.make_async_copy(
                    table.at[pl.ds(0, _B2 * K)], rows[u], sems[u]).wait()
                for q in range(_B2):
                    def cbody(c, cc, q=q, rbuf=rows[u]):
                        sl = pl.ds(c * 16, 16)
                        acc = rbuf[q * K, sl]
                        for r in range(1, K):
                            acc = jnp.maximum(acc, rbuf[q * K + r, sl])
                        outb[q, sl] = acc
                        return cc
                    lax.fori_loop(0, D // 16, cbody, 0)
                pltpu.sync_copy(outb, out.at[pl.ds(wid * _QPW + b * _B2, _B2)])
                if issue_nb is not None:
                    start(issue_nb, u)

            start(0, 0)
            start(1, 1)

            def pbody(p, carry):
                for u in range(2):
                    b = 2 * p + u
                    do_batch(b, u, b + 2)
                return carry

            lax.fori_loop(0, _NB2 // 2 - 1, pbody, 0)
            for u in range(2):
                do_batch(_NB2 - 2 + u, u, None)

    return seg


def _pre_body(x8_ref, w_ref, b_ref, p_ref, q_ref):
    r = jnp.dot(x8_ref[:, :], w_ref[:, :], preferred_element_type=jnp.float32)
    p_ref[:, :] = r[:, :C] + b_ref[:, :]
    q_ref[:, :] = r[:, C:]


_pre_call = pl.pallas_call(
    _pre_body,
    grid=(NPAD // BN,),
    in_specs=[
        pl.BlockSpec((BN, 8), lambda i: (i, 0)),
        pl.BlockSpec((8, 2 * C), lambda i: (0, 0)),
        pl.BlockSpec((1, C), lambda i: (0, 0)),
    ],
    out_specs=[
        pl.BlockSpec((BN, C), lambda i: (i, 0)),
        pl.BlockSpec((BN, C), lambda i: (i, 0)),
    ],
    out_shape=[
        jax.ShapeDtypeStruct((NPAD, C), jnp.float32),
        jax.ShapeDtypeStruct((NPAD, C), jnp.float32),
    ],
)


def _mid_body(pp_ref, sp_ref, x8_ref, wbig_ref, wg1p_ref, bbig_ref,
              p0_ref, q0_ref, p1_ref, q1_ref, g_ref):
    z = pp_ref[:, :] + sp_ref[:, :]
    h = jnp.where(z >= 0, z, 0.2 * z)
    r = jnp.dot(h, wbig_ref[:, :], preferred_element_type=jnp.float32)
    r = r + bbig_ref[:, :]
    rp = jnp.dot(x8_ref[:, :], wg1p_ref[:, :],
                 preferred_element_type=jnp.float32)
    F = C * R
    p0_ref[:, :] = r[:, 0:F]
    q0_ref[:, :] = r[:, F:2 * F]
    p1_ref[:, :] = r[:, 2 * F:3 * F]
    q1_ref[:, :] = r[:, 3 * F:4 * F]
    g_ref[:, :] = r[:, 4 * F:5 * F] + rp


_mid_call = pl.pallas_call(
    _mid_body,
    grid=(NPAD // BN,),
    in_specs=[
        pl.BlockSpec((BN, C), lambda i: (i, 0)),
        pl.BlockSpec((BN, C), lambda i: (i, 0)),
        pl.BlockSpec((BN, 8), lambda i: (i, 0)),
        pl.BlockSpec((C, 5 * C * R), lambda i: (0, 0)),
        pl.BlockSpec((8, C * R), lambda i: (0, 0)),
        pl.BlockSpec((1, 5 * C * R), lambda i: (0, 0)),
    ],
    out_specs=[pl.BlockSpec((BN, C * R), lambda i: (i, 0))] * 5,
    out_shape=[jax.ShapeDtypeStruct((NPAD, C * R), jnp.float32)] * 5,
)


def _chunk_dot(x, w_ref):
    return jnp.concatenate(
        [jnp.dot(x[:, r * C:(r + 1) * C], w_ref[:, :],
                 preferred_element_type=jnp.float32) for r in range(R)],
        axis=1)


def _final_body(p0_ref, s0_ref, p1_ref, s1_ref, g_ref,
                wg2_ref, bg2_ref, wr1_ref, br1_ref, wr2_ref, br2_ref,
                out_ref):
    z0 = p0_ref[:, :] + s0_ref[:, :]
    u0 = jnp.where(z0 >= 0, z0, 0.2 * z0)
    z1 = p1_ref[:, :] + s1_ref[:, :]
    u1 = jnp.where(z1 >= 0, z1, 0.2 * z1)
    g2 = _chunk_dot(g_ref[:, :], wg2_ref) + bg2_ref[:, :]
    xo = (u0 + u1 + g2) / 3.0
    y = _chunk_dot(xo, wr1_ref) + br1_ref[:, :]
    y = jnp.where(y >= 0, y, 0.01 * y)
    out_ref[:, :] = _chunk_dot(y, wr2_ref) + br2_ref[:, :]


_final_call = pl.pallas_call(
    _final_body,
    grid=(NPAD // BN,),
    in_specs=[
        pl.BlockSpec((BN, C * R), lambda i: (i, 0)),
        pl.BlockSpec((BN, C * R), lambda i: (i, 0)),
        pl.BlockSpec((BN, C * R), lambda i: (i, 0)),
        pl.BlockSpec((BN, C * R), lambda i: (i, 0)),
        pl.BlockSpec((BN, C * R), lambda i: (i, 0)),
        pl.BlockSpec((C, C), lambda i: (0, 0)),
        pl.BlockSpec((1, C * R), lambda i: (0, 0)),
        pl.BlockSpec((C, C), lambda i: (0, 0)),
        pl.BlockSpec((1, C * R), lambda i: (0, 0)),
        pl.BlockSpec((C, 32), lambda i: (0, 0)),
        pl.BlockSpec((1, 128), lambda i: (0, 0)),
    ],
    out_specs=pl.BlockSpec((BN, 128), lambda i: (i, 0)),
    out_shape=jax.ShapeDtypeStruct((NPAD, 128), jnp.float32),
)


def kernel(x, W_pre, b_pre, W_up0, b_up0, W_up1, b_up1,
           W_g1, b_g1, W_g2, b_g2, W_r1, b_r1, W_r2, b_r2):
    f32 = jnp.float32
    pos8 = jnp.zeros((NPAD, 8), f32).at[:N, :3].set(x)
    sq = jnp.sum(x * x, axis=1)
    sqp = jnp.zeros((NPAD,), f32).at[:N].set(sq)

    nbrs = _knn_call(pos8, pos8.T, sqp[:, None], sqp[None, :])
    idx_a = nbrs[:, :K].reshape(-1)
    idx_b = nbrs[:, 0:KC:2].reshape(-1)

    W1, W2 = W_pre[:3], W_pre[3:]
    wcat = jnp.zeros((8, 2 * C), f32)
    wcat = wcat.at[:3, :C].set(W1 - W2).at[:3, C:].set(W2)
    Ppre, Qpre = _pre_call(pos8, wcat, b_pre[None, :])
    spre = _segmax(C, False)(Qpre, idx_b)

    F = C * R
    wbig = jnp.concatenate([
        W_up0[:C] - W_up0[C:], W_up0[C:],
        W_up1[:C] - W_up1[C:], W_up1[C:],
        W_g1[:C],
    ], axis=1)
    wg1p = jnp.zeros((8, F), f32).at[:3].set(W_g1[C:])
    bbig = jnp.concatenate([
        b_up0, jnp.zeros((F,), f32), b_up1, jnp.zeros((F,), f32), b_g1,
    ])[None, :]
    P0, Q0, P1, Q1, g = _mid_call(Ppre, spre, pos8, wbig, wg1p, bbig)

    s0, s1 = _segmax(C * R, True)(Q0, idx_a, Q1, idx_b)

    wr2 = jnp.zeros((C, 32), f32).at[:, :3].set(W_r2)
    br2 = jnp.zeros((4, 32), f32).at[:, :3].set(b_r2[None, :]).reshape(-1)
    out = _final_call(P0, s0, P1, s1, g,
                      W_g2, jnp.tile(b_g2, 4)[None, :],
                      W_r1, jnp.tile(b_r1, 4)[None, :],
                      wr2, br2[None, :])
    return out[:N].reshape(N, 4, 32)[:, :, :3].reshape(N * R, 3)


# speedup vs baseline: 1.2320x; 1.2320x over previous
import functools

import jax
import jax.numpy as jnp
from jax import lax
from jax.experimental import pallas as pl
from jax.experimental.pallas import tpu as pltpu
from jax.experimental.pallas import tpu_sc as plsc

N = 10000
C = 128
K = 16
R = 4
KC = 32
NPAD = 10240
BQ = 256
NBLK = NPAD // BQ
BN = 1024

_SC_NC = 2
_SC_NS = 16
_SC_W = _SC_NC * _SC_NS
_QPW = NPAD // _SC_W
_BATCH = 8
_NBATCH = _QPW // _BATCH


_MP = 6
KCX = KC - 1


def _knn_body(q8_ref, posT_ref, sqc_ref, sqr_ref, out_ref, d_ref, v_ref, j_ref):
    i = pl.program_id(0)
    inf = jnp.float32(jnp.inf)
    big = jnp.int32(0x7FFFFFFF)
    d = sqc_ref[:, :] + sqr_ref[:, :] - 2.0 * jnp.dot(
        q8_ref[:, :], posT_ref[:, :], preferred_element_type=jnp.float32)
    jidx = lax.broadcasted_iota(jnp.int32, (BQ, NPAD), 1)
    qidx = i * BQ + lax.broadcasted_iota(jnp.int32, (BQ, NPAD), 0)
    d_ref[:, :] = jnp.where((jidx == qidx) | (jidx >= N), inf, d)

    lane = lax.broadcasted_iota(jnp.int32, (BQ, 128), 1)
    vals = [jnp.full((BQ, 128), inf, jnp.float32) for _ in range(_MP)]
    js = [jnp.zeros((BQ, 128), jnp.int32) for _ in range(_MP)]
    for t in range(NPAD // 128):
        v = d_ref[:, t * 128:(t + 1) * 128]
        j = t * 128 + lane
        for lvl in range(_MP):
            c = v < vals[lvl]
            nv = jnp.where(c, v, vals[lvl])
            pv = jnp.where(c, vals[lvl], v)
            nj = jnp.where(c, j, js[lvl])
            pj = jnp.where(c, js[lvl], j)
            vals[lvl], js[lvl], v, j = nv, nj, pv, pj
    v_ref[:, :] = jnp.concatenate(vals, axis=1)
    j_ref[:, :] = jnp.concatenate(js, axis=1)
    vlast = vals[_MP - 1]

    v32 = None
    for k in range(KCX):
        V = v_ref[:, :]
        Jv = j_ref[:, :]
        m = jnp.min(V, axis=1)
        am = jnp.min(jnp.where(V == m[:, None], Jv, big), axis=1)
        out_ref[:, k] = am
        v_ref[:, :] = jnp.where(Jv == am[:, None], inf, V)
        if k == KCX - 1:
            v32 = m

    anyfail = jnp.any(vlast <= v32[:, None])

    @pl.when(anyfail)
    def _fallback():
        for k in range(KCX):
            dcur = d_ref[:, :]
            m = jnp.min(dcur, axis=1)
            am = jnp.min(jnp.where(dcur == m[:, None], jidx, big), axis=1)
            out_ref[:, k] = am
            d_ref[:, :] = jnp.where(jidx == am[:, None], inf, dcur)


_knn_call = pl.pallas_call(
    _knn_body,
    grid=(NBLK,),
    in_specs=[
        pl.BlockSpec((BQ, 8), lambda i: (i, 0)),
        pl.BlockSpec((8, NPAD), lambda i: (0, 0)),
        pl.BlockSpec((BQ, 1), lambda i: (i, 0)),
        pl.BlockSpec((1, NPAD), lambda i: (0, 0)),
    ],
    out_specs=pl.BlockSpec((BQ, KC), lambda i: (i, 0)),
    out_shape=jax.ShapeDtypeStruct((NPAD, KC), jnp.int32),
    scratch_shapes=[
        pltpu.VMEM((BQ, NPAD), jnp.float32),
        pltpu.VMEM((BQ, _MP * 128), jnp.float32),
        pltpu.VMEM((BQ, _MP * 128), jnp.int32),
    ],
)


_B2 = 4
_NB2 = _QPW // _B2


@functools.lru_cache(maxsize=None)
def _segmax(D, two):
    mesh = plsc.VectorSubcoreMesh(core_axis_name="c", subcore_axis_name="s")
    n_out = 2 if two else 1
    out_type = [jax.ShapeDtypeStruct((NPAD, D), jnp.float32)] * n_out

    @functools.partial(
        pl.kernel,
        out_type=out_type if two else out_type[0],
        mesh=mesh,
        scratch_types=[
            pltpu.VMEM((_QPW * K,), jnp.int32),
            pltpu.VMEM((_B2 * K, D), jnp.float32),
            pltpu.VMEM((_B2 * K, D), jnp.float32),
            pltpu.VMEM((_B2, D), jnp.float32),
            pltpu.SemaphoreType.DMA,
            pltpu.SemaphoreType.DMA,
        ],
    )
    def seg(*refs):
        if two:
            t0, i0, t1, i1, o0, o1, idxw, r0, r1, outb, s0, s1 = refs
            work = ((t0, i0, o0), (t1, i1, o1))
        else:
            t0, i0, o0, idxw, r0, r1, outb, s0, s1 = refs
            work = ((t0, i0, o0),)
        rows = (r0, r1)
        sems = (s0, s1)
        wid = lax.axis_index("s") * _SC_NC + lax.axis_index("c")

        for table, idx, out in work:
            pltpu.sync_copy(idx.at[pl.ds(wid * _QPW * K, _QPW * K)], idxw)

            def start(b, u):
                pltpu.async_copy(
                    table.at[idxw.at[pl.ds(b * (_B2 * K), _B2 * K)]],
                    rows[u], sems[u])

            def do_batch(b, u, issue_nb):
                pltpu.make_async_copy(
                    table.at[pl.ds(0, _B2 * K)], rows[u], sems[u]).wait()
                for q in range(_B2):
                    def cbody(c, cc, q=q, rbuf=rows[u]):
                        sl = pl.ds(c * 16, 16)
                        acc = rbuf[q * K, sl]
                        for r in range(1, K):
                            acc = jnp.maximum(acc, rbuf[q * K + r, sl])
                        outb[q, sl] = acc
                        return cc
                    lax.fori_loop(0, D // 16, cbody, 0)
                pltpu.sync_copy(outb, out.at[pl.ds(wid * _QPW + b * _B2, _B2)])
                if issue_nb is not None:
                    start(issue_nb, u)

            start(0, 0)
            start(1, 1)

            def pbody(p, carry):
                for u in range(2):
                    b = 2 * p + u
                    do_batch(b, u, b + 2)
                return carry

            lax.fori_loop(0, _NB2 // 2 - 1, pbody, 0)
            for u in range(2):
                do_batch(_NB2 - 2 + u, u, None)

    return seg


def _pre_body(x8_ref, w_ref, b_ref, p_ref, q_ref):
    r = jnp.dot(x8_ref[:, :], w_ref[:, :], preferred_element_type=jnp.float32)
    p_ref[:, :] = r[:, :C] + b_ref[:, :]
    q_ref[:, :] = r[:, C:]


_pre_call = pl.pallas_call(
    _pre_body,
    grid=(NPAD // BN,),
    in_specs=[
        pl.BlockSpec((BN, 8), lambda i: (i, 0)),
        pl.BlockSpec((8, 2 * C), lambda i: (0, 0)),
        pl.BlockSpec((1, C), lambda i: (0, 0)),
    ],
    out_specs=[
        pl.BlockSpec((BN, C), lambda i: (i, 0)),
        pl.BlockSpec((BN, C), lambda i: (i, 0)),
    ],
    out_shape=[
        jax.ShapeDtypeStruct((NPAD, C), jnp.float32),
        jax.ShapeDtypeStruct((NPAD, C), jnp.float32),
    ],
)


def _mid_body(pp_ref, sp_ref, x8_ref, wbig_ref, wg1p_ref, bbig_ref,
              p0_ref, q0_ref, p1_ref, q1_ref, g_ref):
    z = pp_ref[:, :] + sp_ref[:, :]
    h = jnp.where(z >= 0, z, 0.2 * z)
    r = jnp.dot(h, wbig_ref[:, :], preferred_element_type=jnp.float32)
    r = r + bbig_ref[:, :]
    rp = jnp.dot(x8_ref[:, :], wg1p_ref[:, :],
                 preferred_element_type=jnp.float32)
    F = C * R
    p0_ref[:, :] = r[:, 0:F]
    q0_ref[:, :] = r[:, F:2 * F]
    p1_ref[:, :] = r[:, 2 * F:3 * F]
    q1_ref[:, :] = r[:, 3 * F:4 * F]
    g_ref[:, :] = r[:, 4 * F:5 * F] + rp


_mid_call = pl.pallas_call(
    _mid_body,
    grid=(NPAD // BN,),
    in_specs=[
        pl.BlockSpec((BN, C), lambda i: (i, 0)),
        pl.BlockSpec((BN, C), lambda i: (i, 0)),
        pl.BlockSpec((BN, 8), lambda i: (i, 0)),
        pl.BlockSpec((C, 5 * C * R), lambda i: (0, 0)),
        pl.BlockSpec((8, C * R), lambda i: (0, 0)),
        pl.BlockSpec((1, 5 * C * R), lambda i: (0, 0)),
    ],
    out_specs=[pl.BlockSpec((BN, C * R), lambda i: (i, 0))] * 5,
    out_shape=[jax.ShapeDtypeStruct((NPAD, C * R), jnp.float32)] * 5,
)


def _chunk_dot(x, w_ref):
    return jnp.concatenate(
        [jnp.dot(x[:, r * C:(r + 1) * C], w_ref[:, :],
                 preferred_element_type=jnp.float32) for r in range(R)],
        axis=1)


def _final_body(p0_ref, s0_ref, p1_ref, s1_ref, g_ref,
                wg2_ref, bg2_ref, wr1_ref, br1_ref, wr2_ref, br2_ref,
                out_ref):
    z0 = p0_ref[:, :] + s0_ref[:, :]
    u0 = jnp.where(z0 >= 0, z0, 0.2 * z0)
    z1 = p1_ref[:, :] + s1_ref[:, :]
    u1 = jnp.where(z1 >= 0, z1, 0.2 * z1)
    g2 = _chunk_dot(g_ref[:, :], wg2_ref) + bg2_ref[:, :]
    xo = (u0 + u1 + g2) / 3.0
    y = _chunk_dot(xo, wr1_ref) + br1_ref[:, :]
    y = jnp.where(y >= 0, y, 0.01 * y)
    out_ref[:, :] = _chunk_dot(y, wr2_ref) + br2_ref[:, :]


_final_call = pl.pallas_call(
    _final_body,
    grid=(NPAD // BN,),
    in_specs=[
        pl.BlockSpec((BN, C * R), lambda i: (i, 0)),
        pl.BlockSpec((BN, C * R), lambda i: (i, 0)),
        pl.BlockSpec((BN, C * R), lambda i: (i, 0)),
        pl.BlockSpec((BN, C * R), lambda i: (i, 0)),
        pl.BlockSpec((BN, C * R), lambda i: (i, 0)),
        pl.BlockSpec((C, C), lambda i: (0, 0)),
        pl.BlockSpec((1, C * R), lambda i: (0, 0)),
        pl.BlockSpec((C, C), lambda i: (0, 0)),
        pl.BlockSpec((1, C * R), lambda i: (0, 0)),
        pl.BlockSpec((C, 32), lambda i: (0, 0)),
        pl.BlockSpec((1, 128), lambda i: (0, 0)),
    ],
    out_specs=pl.BlockSpec((BN, 128), lambda i: (i, 0)),
    out_shape=jax.ShapeDtypeStruct((NPAD, 128), jnp.float32),
)


def kernel(x, W_pre, b_pre, W_up0, b_up0, W_up1, b_up1,
           W_g1, b_g1, W_g2, b_g2, W_r1, b_r1, W_r2, b_r2):
    f32 = jnp.float32
    pos8 = jnp.zeros((NPAD, 8), f32).at[:N, :3].set(x)
    sq = jnp.sum(x * x, axis=1)
    sqp = jnp.zeros((NPAD,), f32).at[:N].set(sq)

    nbrs = _knn_call(pos8, pos8.T, sqp[:, None], sqp[None, :])
    idx_a = nbrs[:, :K].reshape(-1)
    idx_b = nbrs[:, 0:KC:2].reshape(-1)

    W1, W2 = W_pre[:3], W_pre[3:]
    wcat = jnp.zeros((8, 2 * C), f32)
    wcat = wcat.at[:3, :C].set(W1 - W2).at[:3, C:].set(W2)
    Ppre, Qpre = _pre_call(pos8, wcat, b_pre[None, :])
    spre = _segmax(C, False)(Qpre, idx_b)

    F = C * R
    wbig = jnp.concatenate([
        W_up0[:C] - W_up0[C:], W_up0[C:],
        W_up1[:C] - W_up1[C:], W_up1[C:],
        W_g1[:C],
    ], axis=1)
    wg1p = jnp.zeros((8, F), f32).at[:3].set(W_g1[C:])
    bbig = jnp.concatenate([
        b_up0, jnp.zeros((F,), f32), b_up1, jnp.zeros((F,), f32), b_g1,
    ])[None, :]
    P0, Q0, P1, Q1, g = _mid_call(Ppre, spre, pos8, wbig, wg1p, bbig)

    s0, s1 = _segmax(C * R, True)(Q0, idx_a, Q1, idx_b)

    wr2 = jnp.zeros((C, 32), f32).at[:, :3].set(W_r2)
    br2 = jnp.zeros((4, 32), f32).at[:, :3].set(b_r2[None, :]).reshape(-1)
    out = _final_call(P0, s0, P1, s1, g,
                      W_g2, jnp.tile(b_g2, 4)[None, :],
                      W_r1, jnp.tile(b_r1, 4)[None, :],
                      wr2, br2[None, :])
    return out[:N].reshape(N, 4, 32)[:, :, :3].reshape(N * R, 3)
